# baseline (speedup 1.0000x reference)
import functools

import jax
import jax.numpy as jnp
from jax import lax
from jax.experimental import pallas as pl
from jax.experimental.pallas import tpu as pltpu
from jax.experimental.pallas import tpu_sc as plsc

N_ENTITY = 1000000
DIM = 64
B = 16384
L2_COEF = 0.005

NC = 2
NS = 16
NW = NC * NS
WIN = 256
NWIN = N_ENTITY // WIN
WPW = NWIN // NW
TAIL_LO = NWIN * WIN
NV = (3 * B) // 16
CAP = 2048

_mesh = plsc.VectorSubcoreMesh(core_axis_name="c", subcore_axis_name="s")


@functools.partial(
    pl.kernel,
    out_type=jax.ShapeDtypeStruct((3 * B, DIM), jnp.float32),
    mesh=_mesh,
    compiler_params=pltpu.CompilerParams(needs_layout_passes=False),
    scratch_types=(
        pltpu.VMEM((3 * B // 128, 128), jnp.int32),
        pltpu.VMEM((4, DIM, WIN), jnp.float32),
        pltpu.VMEM((CAP,), jnp.int32),
        pltpu.VMEM((CAP,), jnp.int32),
        pltpu.VMEM((32,), jnp.int32),
        pltpu.VMEM((32,), jnp.int32),
        pltpu.VMEM((8, DIM), jnp.float32),
        pltpu.SMEM((8,), jnp.int32),
        pltpu.SemaphoreType.DMA((4,)),
        pltpu.SemaphoreType.DMA,
    ),
)
def _sc_stage(idx_hbm, embt_hbm, tail_hbm, stage_hbm,
              idx_v, win_buf, he_v, hs_v, esc, ssc, rowbuf,
              smem, wsem, osem):
    w = lax.axis_index("s") * NC + lax.axis_index("c")
    is_last = w == NW - 1

    pltpu.sync_copy(idx_hbm, idx_v)

    iota = lax.iota(jnp.int32, 16)
    lo_w = w * (WPW * WIN)
    hi_w = jnp.where(is_last, N_ENTITY, (w + 1) * (WPW * WIN))
    n_win = jnp.where(is_last, WPW + 2, WPW)

    smem[0] = 0
    smem[1] = 0
    smem[2] = 0

    def _idx_vreg(v):
        return idx_v[lax.div(v, 8), pl.ds(lax.rem(v, 8) * 16, 16)]

    def build_body(v, carry):
        e = _idx_vreg(v)
        mask = (e >= lo_w) & (e < hi_w)
        cnt = plsc.all_reduce_population_count(mask)[0]

        @pl.when(cnt > 0)
        def _():
            hc = smem[0]

            @pl.when(hc <= CAP - 16)
            def _():
                plsc.store_compressed(he_v.at[pl.ds(hc, 16)], e, mask=mask)
                plsc.store_compressed(
                    hs_v.at[pl.ds(hc, 16)], v * 16 + iota, mask=mask
                )

            @pl.when(hc > CAP - 16)
            def _():
                smem[1] = 1
            smem[0] = hc + cnt
        return carry

    lax.fori_loop(0, NV, build_body, 0)

    def extract(par, lo, cnt):
        def ebody(j, carry):
            e = esc[pl.ds(j, 16)][0]
            b = ssc[pl.ds(j, 16)][0]
            el = e - lo
            pv = jnp.full((16,), par, jnp.int32)
            elv = jnp.full((16,), el, jnp.int32)
            rc = smem[2]
            rr = lax.rem(rc, 8)

            @pl.when(rc >= 8)
            def _():
                pltpu.make_async_copy(
                    rowbuf.at[pl.ds(0, 1)],
                    stage_hbm.at[pl.ds(0, 1)],
                    osem,
                ).wait()

            for c in range(DIM // 16):
                vec = plsc.load_gather(win_buf, [pv, iota + c * 16, elv])
                rowbuf[rr, pl.ds(c * 16, 16)] = vec
            pltpu.async_copy(
                rowbuf.at[pl.ds(rr, 1)],
                stage_hbm.at[pl.ds(b, 1)],
                osem,
            )
            smem[2] = rc + 1
            return carry

        lax.fori_loop(0, cnt, ebody, 0)

    def process(par, lo, hi):
        ov = smem[1]

        @pl.when(ov == 0)
        def _():
            H = smem[0]
            nhv = lax.div(H + 15, 16)

            def fbody(v, carry):
                e = he_v[pl.ds(v * 16, 16)]
                s = hs_v[pl.ds(v * 16, 16)]
                mask = (v * 16 + iota < H) & (e >= lo) & (e < hi)
                cnt = plsc.all_reduce_population_count(mask)[0]

                @pl.when(cnt > 0)
                def _():
                    plsc.store_compressed(esc.at[pl.ds(0, 16)], e, mask=mask)
                    plsc.store_compressed(ssc.at[pl.ds(0, 16)], s, mask=mask)
                    extract(par, lo, cnt)
                return carry

            lax.fori_loop(0, nhv, fbody, 0)

        @pl.when(ov != 0)
        def _():
            def sbody(v, carry):
                e = _idx_vreg(v)
                mask = (e >= lo) & (e < hi)
                cnt = plsc.all_reduce_population_count(mask)[0]

                @pl.when(cnt > 0)
                def _():
                    plsc.store_compressed(esc.at[pl.ds(0, 16)], e, mask=mask)
                    plsc.store_compressed(
                        ssc.at[pl.ds(0, 16)], v * 16 + iota, mask=mask
                    )
                    extract(par, lo, cnt)
                return carry

            lax.fori_loop(0, NV, sbody, 0)

    @pl.when(is_last)
    def _():
        pltpu.sync_copy(tail_hbm, win_buf.at[3])
        process(3, TAIL_LO, N_ENTITY)

    def fire(s, par):
        pltpu.async_copy(
            embt_hbm.at[:, pl.ds((w * WPW + s) * WIN, WIN)],
            win_buf.at[par],
            wsem.at[par],
        )

    def drainw(par):
        pltpu.make_async_copy(
            embt_hbm.at[:, pl.ds(0, WIN)],
            win_buf.at[par],
            wsem.at[par],
        ).wait()

    for s0 in range(3):
        fire(s0, s0)

    def wbody(s, carry):
        par = lax.rem(s, 4)

        @pl.when(s + 3 < n_win)
        def _():
            fire(s + 3, lax.rem(s + 3, 4))

        @pl.when(s < n_win)
        def _():
            drainw(par)
            lo = (w * WPW + s) * WIN
            process(par, lo, lo + WIN)
        return carry

    lax.fori_loop(0, WPW + 2, wbody, 0)

    rc = smem[2]
    n_drain = jnp.minimum(rc, 8)

    def dbody(j, carry):
        pltpu.make_async_copy(
            rowbuf.at[pl.ds(0, 1)],
            stage_hbm.at[pl.ds(0, 1)],
            osem,
        ).wait()
        return carry

    lax.fori_loop(0, n_drain, dbody, 0)


def _tc_body(h_ref, p_ref, n_ref, out_ref):
    i = pl.program_id(0)
    hh = h_ref[...]
    pp = p_ref[...]
    nn = n_ref[...]
    d = jnp.sum(hh * (nn - pp), axis=1, keepdims=True)
    z = -d
    sp = jnp.maximum(z, 0.0) + jnp.log1p(jnp.exp(-jnp.abs(z)))
    kg = jnp.sum(sp)
    sq = jnp.sum(hh * hh) + jnp.sum(pp * pp) + jnp.sum(nn * nn)
    contrib = kg * (1.0 / B) + (L2_COEF * 0.5 / B) * sq

    @pl.when(i == 0)
    def _():
        out_ref[0, 0] = 0.0

    out_ref[0, 0] += contrib


_tc_finish = pl.pallas_call(
    _tc_body,
    grid=(16,),
    in_specs=[
        pl.BlockSpec((1024, DIM), lambda i: (i, 0)),
        pl.BlockSpec((1024, DIM), lambda i: (i + 16, 0)),
        pl.BlockSpec((1024, DIM), lambda i: (i + 32, 0)),
    ],
    out_specs=pl.BlockSpec((1, 1), lambda i: (0, 0), memory_space=pltpu.SMEM),
    out_shape=jax.ShapeDtypeStruct((1, 1), jnp.float32),
)


def kernel(h, pos_t, neg_t, emb):
    idx_all = jnp.concatenate(
        [h.astype(jnp.int32), pos_t.astype(jnp.int32), neg_t.astype(jnp.int32)]
    ).reshape(3 * B // 128, 128)
    embt = emb.T
    tail = jnp.pad(embt[:, TAIL_LO:], ((0, 0), (0, WIN - (N_ENTITY - TAIL_LO))))
    stage = _sc_stage(idx_all, embt, tail)
    loss = _tc_finish(stage, stage, stage)
    return loss[0, 0]


# speedup vs baseline: 1.8056x; 1.8056x over previous
import functools

import jax
import jax.numpy as jnp
from jax import lax
from jax.experimental import pallas as pl
from jax.experimental.pallas import tpu as pltpu
from jax.experimental.pallas import tpu_sc as plsc

N_ENTITY = 1000000
DIM = 64
B = 16384
L2_COEF = 0.005

NC = 2
NS = 16
NW = NC * NS
WIN = 512
NWIN = N_ENTITY // WIN
WPW = NWIN // NW
EXTRA = NWIN - NW * WPW
TAIL_LO = NWIN * WIN
NV = (3 * B) // 16
CAP = 2048

_mesh = plsc.VectorSubcoreMesh(core_axis_name="c", subcore_axis_name="s")


@functools.partial(
    pl.kernel,
    out_type=jax.ShapeDtypeStruct((3 * B, DIM), jnp.float32),
    mesh=_mesh,
    compiler_params=pltpu.CompilerParams(needs_layout_passes=False),
    scratch_types=(
        pltpu.VMEM((3 * B // 128, 128), jnp.int32),
        pltpu.VMEM((2, DIM, WIN), jnp.float32),
        pltpu.VMEM((CAP,), jnp.int32),
        pltpu.VMEM((CAP,), jnp.int32),
        pltpu.VMEM((2080,), jnp.int32),
        pltpu.VMEM((2080,), jnp.int32),
        pltpu.VMEM((32,), jnp.int32),
        pltpu.VMEM((32,), jnp.int32),
        pltpu.VMEM((8, DIM), jnp.float32),
        pltpu.SMEM((8,), jnp.int32),
        pltpu.SemaphoreType.DMA((2,)),
        pltpu.SemaphoreType.DMA,
    ),
)
def _sc_stage(idx_hbm, embt_hbm, tail_hbm, stage_hbm,
              idx_v, win_buf, he_v, hs_v, sle_v, sls_v, esc, ssc, rowbuf,
              smem, wsem, osem):
    w = lax.axis_index("s") * NC + lax.axis_index("c")
    is_last = w == NW - 1

    pltpu.sync_copy(idx_hbm, idx_v)

    iota = lax.iota(jnp.int32, 16)
    lo_w = w * (WPW * WIN)
    hi_w = jnp.where(is_last, N_ENTITY, (w + 1) * (WPW * WIN))
    n_win = jnp.where(is_last, WPW + EXTRA, WPW)

    smem[0] = 0
    smem[1] = 0
    smem[2] = 0

    def _idx_vreg(v):
        return idx_v[lax.div(v, 8), pl.ds(lax.rem(v, 8) * 16, 16)]

    def build_body(v, carry):
        e = _idx_vreg(v)
        mask = (e >= lo_w) & (e < hi_w)

        @pl.when(jnp.any(mask))
        def _():
            cnt = plsc.all_reduce_population_count(mask)[0]
            hc = smem[0]

            @pl.when(hc <= CAP - 16)
            def _():
                plsc.store_compressed(he_v.at[pl.ds(hc, 16)], e, mask=mask)
                plsc.store_compressed(
                    hs_v.at[pl.ds(hc, 16)], v * 16 + iota, mask=mask
                )

            @pl.when(hc > CAP - 16)
            def _():
                smem[1] = 1
            smem[0] = hc + cnt
        return carry

    lax.fori_loop(0, NV, build_body, 0)

    def extract(par, lo, cnt):
        def ebody(j, carry):
            e = esc[pl.ds(j, 16)][0]
            b = ssc[pl.ds(j, 16)][0]
            el = e - lo
            pv = jnp.full((16,), par, jnp.int32)
            elv = jnp.full((16,), el, jnp.int32)
            rc = smem[2]
            rr = lax.rem(rc, 8)

            @pl.when(rc >= 8)
            def _():
                pltpu.make_async_copy(
                    rowbuf.at[pl.ds(0, 1)],
                    stage_hbm.at[pl.ds(0, 1)],
                    osem,
                ).wait()

            for c in range(DIM // 16):
                vec = plsc.load_gather(win_buf, [pv, iota + c * 16, elv])
                rowbuf[rr, pl.ds(c * 16, 16)] = vec
            pltpu.async_copy(
                rowbuf.at[pl.ds(rr, 1)],
                stage_hbm.at[pl.ds(b, 1)],
                osem,
            )
            smem[2] = rc + 1
            return carry

        lax.fori_loop(0, cnt, ebody, 0)

    def build_super(slo, shi):
        smem[3] = 0
        H = smem[0]
        nhv = lax.div(H + 15, 16)

        def bsbody(v, carry):
            e = he_v[pl.ds(v * 16, 16)]
            s = hs_v[pl.ds(v * 16, 16)]
            mask = (v * 16 + iota < H) & (e >= slo) & (e < shi)

            @pl.when(jnp.any(mask))
            def _():
                cnt = plsc.all_reduce_population_count(mask)[0]
                hs2 = smem[3]
                plsc.store_compressed(sle_v.at[pl.ds(hs2, 16)], e, mask=mask)
                plsc.store_compressed(sls_v.at[pl.ds(hs2, 16)], s, mask=mask)
                smem[3] = hs2 + cnt
            return carry

        lax.fori_loop(0, nhv, bsbody, 0)

    def process_fast(par, lo, hi):
        Hs = smem[3]
        nhv = lax.div(Hs + 15, 16)

        def fbody(v, carry):
            e = sle_v[pl.ds(v * 16, 16)]
            s = sls_v[pl.ds(v * 16, 16)]
            mask = (v * 16 + iota < Hs) & (e >= lo) & (e < hi)

            @pl.when(jnp.any(mask))
            def _():
                cnt = plsc.all_reduce_population_count(mask)[0]
                plsc.store_compressed(esc.at[pl.ds(0, 16)], e, mask=mask)
                plsc.store_compressed(ssc.at[pl.ds(0, 16)], s, mask=mask)
                extract(par, lo, cnt)
            return carry

        lax.fori_loop(0, nhv, fbody, 0)

    def process_slow(par, lo, hi):
        def sbody(v, carry):
            e = _idx_vreg(v)
            mask = (e >= lo) & (e < hi)

            @pl.when(jnp.any(mask))
            def _():
                cnt = plsc.all_reduce_population_count(mask)[0]
                plsc.store_compressed(esc.at[pl.ds(0, 16)], e, mask=mask)
                plsc.store_compressed(
                    ssc.at[pl.ds(0, 16)], v * 16 + iota, mask=mask
                )
                extract(par, lo, cnt)
            return carry

        lax.fori_loop(0, NV, sbody, 0)

    def process(par, lo, hi):
        ov = smem[1]

        @pl.when(ov == 0)
        def _():
            smem[3] = 0
            build_super(lo, hi)
            process_fast(par, lo, hi)

        @pl.when(ov != 0)
        def _():
            process_slow(par, lo, hi)

    @pl.when(is_last)
    def _():
        pltpu.sync_copy(tail_hbm, win_buf.at[1])
        process(1, TAIL_LO, N_ENTITY)

    def fire(s, par):
        pltpu.async_copy(
            embt_hbm.at[:, pl.ds((w * WPW + s) * WIN, WIN)],
            win_buf.at[par],
            wsem.at[par],
        )

    def drainw(par):
        pltpu.make_async_copy(
            embt_hbm.at[:, pl.ds(0, WIN)],
            win_buf.at[par],
            wsem.at[par],
        ).wait()

    for s0 in range(2):
        fire(s0, s0)

    ov_g = smem[1]

    def wsuper(s2, carry):
        sn = jnp.clip(n_win - s2 * 8, 0, 8)
        slo = (w * WPW + s2 * 8) * WIN
        shi = slo + sn * WIN

        @pl.when((sn > 0) & (ov_g == 0))
        def _():
            build_super(slo, shi)

        def wbody(j, carry2):
            s = s2 * 8 + j
            par = lax.rem(s, 2)

            @pl.when(s < n_win)
            def _():
                drainw(par)
                lo = (w * WPW + s) * WIN

                @pl.when(ov_g == 0)
                def _():
                    process_fast(par, lo, lo + WIN)

                @pl.when(ov_g != 0)
                def _():
                    process_slow(par, lo, lo + WIN)

                @pl.when(s + 2 < n_win)
                def _():
                    fire(s + 2, par)
            return carry2

        lax.fori_loop(0, 8, wbody, 0)
        return carry

    lax.fori_loop(0, (WPW + EXTRA + 7) // 8, wsuper, 0)

    rc = smem[2]
    n_drain = jnp.minimum(rc, 8)

    def dbody(j, carry):
        pltpu.make_async_copy(
            rowbuf.at[pl.ds(0, 1)],
            stage_hbm.at[pl.ds(0, 1)],
            osem,
        ).wait()
        return carry

    lax.fori_loop(0, n_drain, dbody, 0)


def _tc_body(h_ref, p_ref, n_ref, out_ref):
    i = pl.program_id(0)
    hh = h_ref[...]
    pp = p_ref[...]
    nn = n_ref[...]
    d = jnp.sum(hh * (nn - pp), axis=1, keepdims=True)
    z = -d
    sp = jnp.maximum(z, 0.0) + jnp.log1p(jnp.exp(-jnp.abs(z)))
    kg = jnp.sum(sp)
    sq = jnp.sum(hh * hh) + jnp.sum(pp * pp) + jnp.sum(nn * nn)
    contrib = kg * (1.0 / B) + (L2_COEF * 0.5 / B) * sq

    @pl.when(i == 0)
    def _():
        out_ref[0, 0] = 0.0

    out_ref[0, 0] += contrib


_tc_finish = pl.pallas_call(
    _tc_body,
    grid=(16,),
    in_specs=[
        pl.BlockSpec((1024, DIM), lambda i: (i, 0)),
        pl.BlockSpec((1024, DIM), lambda i: (i + 16, 0)),
        pl.BlockSpec((1024, DIM), lambda i: (i + 32, 0)),
    ],
    out_specs=pl.BlockSpec((1, 1), lambda i: (0, 0), memory_space=pltpu.SMEM),
    out_shape=jax.ShapeDtypeStruct((1, 1), jnp.float32),
)


def kernel(h, pos_t, neg_t, emb):
    idx_all = jnp.concatenate(
        [h.astype(jnp.int32), pos_t.astype(jnp.int32), neg_t.astype(jnp.int32)]
    ).reshape(3 * B // 128, 128)
    embt = emb.T
    tail = jnp.pad(embt[:, TAIL_LO:], ((0, 0), (0, WIN - (N_ENTITY - TAIL_LO))))
    stage = _sc_stage(idx_all, embt, tail)
    loss = _tc_finish(stage, stage, stage)
    return loss[0, 0]


# speedup vs baseline: 1.8161x; 1.0058x over previous
import functools

import jax
import jax.numpy as jnp
from jax import lax
from jax.experimental import pallas as pl
from jax.experimental.pallas import tpu as pltpu
from jax.experimental.pallas import tpu_sc as plsc

N_ENTITY = 1000000
DIM = 64
B = 16384
L2_COEF = 0.005

NC = 2
NS = 16
NW = NC * NS
WIN = 512
NWIN = N_ENTITY // WIN
WPW = NWIN // NW
EXTRA = NWIN - NW * WPW
TAIL_LO = NWIN * WIN
NV = (3 * B) // 16
CAP = 2048

_mesh = plsc.VectorSubcoreMesh(core_axis_name="c", subcore_axis_name="s")


@functools.partial(
    pl.kernel,
    out_type=jax.ShapeDtypeStruct((3 * B, DIM), jnp.float32),
    mesh=_mesh,
    compiler_params=pltpu.CompilerParams(needs_layout_passes=False),
    scratch_types=(
        pltpu.VMEM((3 * B // 128, 128), jnp.int32),
        pltpu.VMEM((2, DIM, WIN), jnp.float32),
        pltpu.VMEM((CAP,), jnp.int32),
        pltpu.VMEM((CAP,), jnp.int32),
        pltpu.VMEM((2080,), jnp.int32),
        pltpu.VMEM((2080,), jnp.int32),
        pltpu.VMEM((32,), jnp.int32),
        pltpu.VMEM((32,), jnp.int32),
        pltpu.VMEM((8, DIM), jnp.float32),
        pltpu.SMEM((8,), jnp.int32),
        pltpu.SemaphoreType.DMA((2,)),
        pltpu.SemaphoreType.DMA,
    ),
)
def _sc_stage(idx_hbm, embt_hbm, tail_hbm, stage_hbm,
              idx_v, win_buf, he_v, hs_v, sle_v, sls_v, esc, ssc, rowbuf,
              smem, wsem, osem):
    w = lax.axis_index("s") * NC + lax.axis_index("c")
    is_last = w == NW - 1

    pltpu.sync_copy(idx_hbm, idx_v)

    def fire(s, par):
        pltpu.async_copy(
            embt_hbm.at[:, pl.ds((w * WPW + s) * WIN, WIN)],
            win_buf.at[par],
            wsem.at[par],
        )

    for s0 in range(2):
        fire(s0, s0)

    iota = lax.iota(jnp.int32, 16)
    lo_w = w * (WPW * WIN)
    hi_w = jnp.where(is_last, N_ENTITY, (w + 1) * (WPW * WIN))
    n_win = jnp.where(is_last, WPW + EXTRA, WPW)

    smem[0] = 0
    smem[1] = 0
    smem[2] = 0

    def _idx_vreg(v):
        return idx_v[lax.div(v, 8), pl.ds(lax.rem(v, 8) * 16, 16)]

    def build_body(v, carry):
        e = _idx_vreg(v)
        mask = (e >= lo_w) & (e < hi_w)

        @pl.when(jnp.any(mask))
        def _():
            cnt = plsc.all_reduce_population_count(mask)[0]
            hc = smem[0]

            @pl.when(hc <= CAP - 16)
            def _():
                plsc.store_compressed(he_v.at[pl.ds(hc, 16)], e, mask=mask)
                plsc.store_compressed(
                    hs_v.at[pl.ds(hc, 16)], v * 16 + iota, mask=mask
                )

            @pl.when(hc > CAP - 16)
            def _():
                smem[1] = 1
            smem[0] = hc + cnt
        return carry

    lax.fori_loop(0, NV, build_body, 0)

    def extract(par, lo, cnt):
        def ebody(j, carry):
            e = esc[pl.ds(j, 16)][0]
            b = ssc[pl.ds(j, 16)][0]
            el = e - lo
            pv = jnp.full((16,), par, jnp.int32)
            elv = jnp.full((16,), el, jnp.int32)
            rc = smem[2]
            rr = lax.rem(rc, 8)

            @pl.when(rc >= 8)
            def _():
                pltpu.make_async_copy(
                    rowbuf.at[pl.ds(0, 1)],
                    stage_hbm.at[pl.ds(0, 1)],
                    osem,
                ).wait()

            for c in range(DIM // 16):
                vec = plsc.load_gather(win_buf, [pv, iota + c * 16, elv])
                rowbuf[rr, pl.ds(c * 16, 16)] = vec
            pltpu.async_copy(
                rowbuf.at[pl.ds(rr, 1)],
                stage_hbm.at[pl.ds(b, 1)],
                osem,
            )
            smem[2] = rc + 1
            return carry

        lax.fori_loop(0, cnt, ebody, 0)

    def build_super(slo, shi):
        smem[3] = 0
        H = smem[0]
        nhv = lax.div(H + 15, 16)

        def bsbody(v, carry):
            e = he_v[pl.ds(v * 16, 16)]
            s = hs_v[pl.ds(v * 16, 16)]
            mask = (v * 16 + iota < H) & (e >= slo) & (e < shi)

            @pl.when(jnp.any(mask))
            def _():
                cnt = plsc.all_reduce_population_count(mask)[0]
                hs2 = smem[3]
                plsc.store_compressed(sle_v.at[pl.ds(hs2, 16)], e, mask=mask)
                plsc.store_compressed(sls_v.at[pl.ds(hs2, 16)], s, mask=mask)
                smem[3] = hs2 + cnt
            return carry

        lax.fori_loop(0, nhv, bsbody, 0)

    def process_fast(par, lo, hi):
        Hs = smem[3]
        nhv = lax.div(Hs + 15, 16)

        def fbody(v, carry):
            e = sle_v[pl.ds(v * 16, 16)]
            s = sls_v[pl.ds(v * 16, 16)]
            mask = (v * 16 + iota < Hs) & (e >= lo) & (e < hi)

            @pl.when(jnp.any(mask))
            def _():
                cnt = plsc.all_reduce_population_count(mask)[0]
                plsc.store_compressed(esc.at[pl.ds(0, 16)], e, mask=mask)
                plsc.store_compressed(ssc.at[pl.ds(0, 16)], s, mask=mask)
                extract(par, lo, cnt)
            return carry

        lax.fori_loop(0, nhv, fbody, 0)

    def process_slow(par, lo, hi):
        def sbody(v, carry):
            e = _idx_vreg(v)
            mask = (e >= lo) & (e < hi)

            @pl.when(jnp.any(mask))
            def _():
                cnt = plsc.all_reduce_population_count(mask)[0]
                plsc.store_compressed(esc.at[pl.ds(0, 16)], e, mask=mask)
                plsc.store_compressed(
                    ssc.at[pl.ds(0, 16)], v * 16 + iota, mask=mask
                )
                extract(par, lo, cnt)
            return carry

        lax.fori_loop(0, NV, sbody, 0)

    def process(par, lo, hi):
        ov = smem[1]

        @pl.when(ov == 0)
        def _():
            smem[3] = 0
            build_super(lo, hi)
            process_fast(par, lo, hi)

        @pl.when(ov != 0)
        def _():
            process_slow(par, lo, hi)

    def drainw(par):
        pltpu.make_async_copy(
            embt_hbm.at[:, pl.ds(0, WIN)],
            win_buf.at[par],
            wsem.at[par],
        ).wait()

    ov_g = smem[1]

    def wsuper(s2, carry):
        sn = jnp.clip(n_win - s2 * 8, 0, 8)
        slo = (w * WPW + s2 * 8) * WIN
        shi = slo + sn * WIN

        @pl.when((sn > 0) & (ov_g == 0))
        def _():
            build_super(slo, shi)

        def wbody(j, carry2):
            s = s2 * 8 + j
            par = lax.rem(s, 2)

            @pl.when(s < n_win)
            def _():
                drainw(par)
                lo = (w * WPW + s) * WIN

                @pl.when(ov_g == 0)
                def _():
                    process_fast(par, lo, lo + WIN)

                @pl.when(ov_g != 0)
                def _():
                    process_slow(par, lo, lo + WIN)

                @pl.when(s + 2 < n_win)
                def _():
                    fire(s + 2, par)
            return carry2

        lax.fori_loop(0, 8, wbody, 0)
        return carry

    lax.fori_loop(0, (WPW + EXTRA + 7) // 8, wsuper, 0)

    @pl.when(is_last)
    def _():
        pltpu.sync_copy(tail_hbm, win_buf.at[0])
        process(0, TAIL_LO, N_ENTITY)

    rc = smem[2]
    n_drain = jnp.minimum(rc, 8)

    def dbody(j, carry):
        pltpu.make_async_copy(
            rowbuf.at[pl.ds(0, 1)],
            stage_hbm.at[pl.ds(0, 1)],
            osem,
        ).wait()
        return carry

    lax.fori_loop(0, n_drain, dbody, 0)


def _tc_body(h_ref, p_ref, n_ref, out_ref):
    i = pl.program_id(0)
    hh = h_ref[...]
    pp = p_ref[...]
    nn = n_ref[...]
    d = jnp.sum(hh * (nn - pp), axis=1, keepdims=True)
    z = -d
    sp = jnp.maximum(z, 0.0) + jnp.log1p(jnp.exp(-jnp.abs(z)))
    kg = jnp.sum(sp)
    sq = jnp.sum(hh * hh) + jnp.sum(pp * pp) + jnp.sum(nn * nn)
    contrib = kg * (1.0 / B) + (L2_COEF * 0.5 / B) * sq

    @pl.when(i == 0)
    def _():
        out_ref[0, 0] = 0.0

    out_ref[0, 0] += contrib


_tc_finish = pl.pallas_call(
    _tc_body,
    grid=(16,),
    in_specs=[
        pl.BlockSpec((1024, DIM), lambda i: (i, 0)),
        pl.BlockSpec((1024, DIM), lambda i: (i + 16, 0)),
        pl.BlockSpec((1024, DIM), lambda i: (i + 32, 0)),
    ],
    out_specs=pl.BlockSpec((1, 1), lambda i: (0, 0), memory_space=pltpu.SMEM),
    out_shape=jax.ShapeDtypeStruct((1, 1), jnp.float32),
)


def kernel(h, pos_t, neg_t, emb):
    idx_all = jnp.concatenate(
        [h.astype(jnp.int32), pos_t.astype(jnp.int32), neg_t.astype(jnp.int32)]
    ).reshape(3 * B // 128, 128)
    embt = emb.T
    tail = jnp.pad(embt[:, TAIL_LO:], ((0, 0), (0, WIN - (N_ENTITY - TAIL_LO))))
    stage = _sc_stage(idx_all, embt, tail)
    loss = _tc_finish(stage, stage, stage)
    return loss[0, 0]
